# baseline (speedup 1.0000x reference)
import math

import jax
import jax.numpy as jnp
from jax.experimental import pallas as pl

SCALE_POSE = 10.0
SCALE_RADAR_V = 5.0
L_MIN = 2 * math.log(0.03 / SCALE_POSE + 1e-09)
L_MAX = 2 * math.log(0.5 / SCALE_POSE + 1e-09)
R_MIN = 2 * math.log(0.1 / SCALE_RADAR_V + 1e-09)
R_MAX = 2 * math.log(5.0 / SCALE_RADAR_V + 1e-09)
RADIUS_CROSS = 0.6 / SCALE_POSE
GHOST = RADIUS_CROSS ** 2


def _scatter_mean(vals, idx, num):
    s = jax.ops.segment_sum(vals, idx, num_segments=num)
    c = jax.ops.segment_sum(
        jnp.ones((vals.shape[0],) + (1,) * (vals.ndim - 1), vals.dtype),
        idx, num_segments=num)
    return s / jnp.maximum(c, 1.0)


def _radar_branch(out, pos, x, batch_idx, temp_ei, src_r, dst_l, lidar_pos,
                  dt_sec, gt_pos):
    lv = jnp.clip(out, R_MIN, R_MAX)
    node_dt = jnp.maximum(dt_sec[batch_idx][:, None], 0.01)
    N = pos.shape[0]
    src, dst = temp_ei[0], temp_ei[1]
    move = pos[dst] - pos[src]
    unit = move / (jnp.linalg.norm(move, axis=1, keepdims=True) + 1e-09)
    speed = jnp.abs(x[src, 2:3])
    pred = pos[src] + speed * unit * node_dt[src]
    a2 = jnp.sum(pred ** 2, axis=1, keepdims=True)
    b2 = jnp.sum(gt_pos ** 2, axis=1)[None, :]
    d2 = jnp.maximum(a2 + b2 - 2.0 * pred @ gt_pos.T, 0.0)
    min_d2 = jnp.min(d2, axis=1)
    E = src.shape[0]
    eid = jnp.arange(E, dtype=jnp.int32)
    last_e = jax.ops.segment_max(eid, src, num_segments=N)
    touched = last_e >= 0
    safe_e = jnp.where(touched, last_e, 0)
    phys = jnp.where(touched, min_d2[safe_e], 0.0)
    physics_err_sq = phys[:, None]
    dist_sq = jnp.sum((pos[src_r] - lidar_pos[dst_l]) ** 2, axis=1)
    sum_d = jax.ops.segment_sum(dist_sq, src_r, num_segments=N)
    cnt_d = jax.ops.segment_sum(jnp.ones_like(dist_sq), src_r, num_segments=N)
    val = (sum_d / jnp.maximum(cnt_d, 1.0) ** 2)[:, None]
    spatial_err_sq = jnp.where(cnt_d[:, None] > 0, val, GHOST)
    denom = 2.0 * jnp.exp(lv) * node_dt ** 2 + 1e-09
    r_temp = jnp.mean(physics_err_sq / denom)
    r_spat = jnp.mean(spatial_err_sq / denom)
    r_reg = jnp.mean(0.5 * lv)
    return r_temp + r_spat + r_reg


def _finalize_kernel(parts_ref, out_ref):
    out_ref[...] = jnp.sum(parts_ref[...], keepdims=True)


def kernel(lidar_out, lidar_pos, lidar_x, lidar_spatial_edge_index,
           radar1_out, radar1_pos, radar1_x, radar1_batch,
           radar1_temporal_edge_index, radar1_to_lidar_src,
           radar1_to_lidar_dst, radar2_out, radar2_pos, radar2_x,
           radar2_batch, radar2_temporal_edge_index, radar2_to_lidar_src,
           radar2_to_lidar_dst, dt_sec, gt_radar_pos):
    lv = jnp.clip(lidar_out, L_MIN, L_MAX)
    curr_int = lidar_x[:, 2:3]
    src, dst = lidar_spatial_edge_index[0], lidar_spatial_edge_index[1]
    N = lidar_pos.shape[0]
    mean_pos = _scatter_mean(lidar_pos[src], dst, N)
    mean_int = _scatter_mean(curr_int[src], dst, N)
    res_pos = jnp.sum((lidar_pos - mean_pos) ** 2, axis=1, keepdims=True)
    res_int = (curr_int - mean_int) ** 2
    precision = jnp.exp(-lv)
    l_spat = jnp.mean(0.5 * precision * res_pos)
    l_int = jnp.mean(0.5 * precision * res_int)
    l_reg = jnp.mean(0.5 * lv)
    t1 = l_spat + l_int + l_reg
    t2 = _radar_branch(radar1_out, radar1_pos, radar1_x, radar1_batch,
                       radar1_temporal_edge_index, radar1_to_lidar_src,
                       radar1_to_lidar_dst, lidar_pos, dt_sec, gt_radar_pos)
    t3 = _radar_branch(radar2_out, radar2_pos, radar2_x, radar2_batch,
                       radar2_temporal_edge_index, radar2_to_lidar_src,
                       radar2_to_lidar_dst, lidar_pos, dt_sec, gt_radar_pos)
    parts = jnp.stack([t1, t2, t3]).reshape(1, 3)
    out = pl.pallas_call(
        _finalize_kernel,
        out_shape=jax.ShapeDtypeStruct((1, 1), jnp.float32),
    )(parts)
    return out[0, 0]


# speedup vs baseline: 4.2879x; 4.2879x over previous
import functools
import math

import jax
import jax.numpy as jnp
from jax import lax
from jax.experimental import pallas as pl
from jax.experimental.pallas import tpu as pltpu
import jax.experimental.pallas.tpu_sc as plsc

SCALE_POSE = 10.0
SCALE_RADAR_V = 5.0
L_MIN = 2 * math.log(0.03 / SCALE_POSE + 1e-09)
L_MAX = 2 * math.log(0.5 / SCALE_POSE + 1e-09)
R_MIN = 2 * math.log(0.1 / SCALE_RADAR_V + 1e-09)
R_MAX = 2 * math.log(5.0 / SCALE_RADAR_V + 1e-09)
RADIUS_CROSS = 0.6 / SCALE_POSE
GHOST = RADIUS_CROSS ** 2

N_LIDAR = 100000
E_LIDAR = 1600000
N_ACC = 102400
NW = 32
NS = 16
LANE = 128
NBATCH = 16
NCHUNK = 25
E_TILE = NCHUNK * NBATCH * LANE
E_PAD = NW * E_TILE
ROWS_PER_TILE = N_ACC // NS


def _lidar_sc_body(table_hbm, src_hbm, dst_hbm, zeros_hbm, out_hbm,
                   idxs_v, idxd_v, rows_v, acc_sh, sem):
    c = lax.axis_index("c")
    s = lax.axis_index("s")
    pltpu.sync_copy(zeros_hbm, acc_sh.at[pl.ds(s * ROWS_PER_TILE,
                                               ROWS_PER_TILE)])
    plsc.subcore_barrier()

    def chunk(ch, _):
        pltpu.sync_copy(src_hbm.at[c, s, ch], idxs_v)
        pltpu.sync_copy(dst_hbm.at[c, s, ch], idxd_v)
        for j in range(NBATCH):
            pltpu.async_copy(table_hbm.at[idxs_v.at[j]], rows_v, sem).wait()
            pltpu.sync_copy(rows_v, acc_sh.at[idxd_v.at[j]], add=True)
        return ()

    lax.fori_loop(0, NCHUNK, chunk, ())
    plsc.subcore_barrier()
    pltpu.sync_copy(acc_sh.at[pl.ds(s * ROWS_PER_TILE, ROWS_PER_TILE)],
                    out_hbm.at[c].at[pl.ds(s * ROWS_PER_TILE, ROWS_PER_TILE)])


@functools.partial(jax.jit, static_argnums=())
def _lidar_sc(table, src_idx, dst_idx, zeros_stripe):
    mesh = plsc.VectorSubcoreMesh(core_axis_name="c", subcore_axis_name="s")
    k = pl.kernel(
        _lidar_sc_body,
        out_type=jax.ShapeDtypeStruct((2, N_ACC, 8), jnp.float32),
        mesh=mesh,
        scratch_types=[
            pltpu.VMEM((NBATCH, LANE), jnp.int32),
            pltpu.VMEM((NBATCH, LANE), jnp.int32),
            pltpu.VMEM((LANE, 8), jnp.float32),
            pltpu.VMEM_SHARED((N_ACC, 8), jnp.float32),
            pltpu.SemaphoreType.DMA,
        ],
        compiler_params=pltpu.CompilerParams(use_tc_tiling_on_sc=False),
    )
    return k(table, src_idx, dst_idx, zeros_stripe)


BLK = 1024
NBLK = N_ACC // BLK


def _lidar_tc_body(acc0_ref, acc1_ref, table_ref, out_ref):
    i = pl.program_id(0)
    a = acc0_ref[...] + acc1_ref[...]
    t = table_ref[...]
    lv = jnp.clip(t[5:6, :], L_MIN, L_MAX)
    c1 = jnp.maximum(a[4:5, :], 1.0)
    mean_pos = a[0:3, :] / c1
    mean_int = a[3:4, :] / c1
    pos = t[0:3, :]
    ci = t[3:4, :]
    valid = t[4:5, :]
    res_pos = jnp.sum((pos - mean_pos) ** 2, axis=0, keepdims=True)
    res_int = (ci - mean_int) ** 2
    prec = jnp.exp(-lv)
    s1 = jnp.sum(valid * 0.5 * prec * res_pos)
    s2 = jnp.sum(valid * 0.5 * prec * res_int)
    s3 = jnp.sum(valid * 0.5 * lv)
    col = lax.broadcasted_iota(jnp.int32, (8, 128), 1)
    row = lax.broadcasted_iota(jnp.int32, (8, 128), 0)
    sel = row == 0
    part = (jnp.where(sel & (col == 0), s1, 0.0)
            + jnp.where(sel & (col == 1), s2, 0.0)
            + jnp.where(sel & (col == 2), s3, 0.0))

    @pl.when(i == 0)
    def _():
        out_ref[...] = jnp.zeros_like(out_ref)

    out_ref[...] += part


def _lidar_tc(acc0T, acc1T, tableT):
    return pl.pallas_call(
        _lidar_tc_body,
        grid=(NBLK,),
        in_specs=[
            pl.BlockSpec((8, BLK), lambda i: (0, i)),
            pl.BlockSpec((8, BLK), lambda i: (0, i)),
            pl.BlockSpec((8, BLK), lambda i: (0, i)),
        ],
        out_specs=pl.BlockSpec((8, 128), lambda i: (0, 0)),
        out_shape=jax.ShapeDtypeStruct((8, 128), jnp.float32),
    )(acc0T, acc1T, tableT)


def _radar_branch(out, pos, x, batch_idx, temp_ei, src_r, dst_l, lidar_pos,
                  dt_sec, gt_pos):
    lv = jnp.clip(out, R_MIN, R_MAX)
    node_dt = jnp.maximum(dt_sec[batch_idx][:, None], 0.01)
    N = pos.shape[0]
    src, dst = temp_ei[0], temp_ei[1]
    move = pos[dst] - pos[src]
    unit = move / (jnp.linalg.norm(move, axis=1, keepdims=True) + 1e-09)
    speed = jnp.abs(x[src, 2:3])
    pred = pos[src] + speed * unit * node_dt[src]
    a2 = jnp.sum(pred ** 2, axis=1, keepdims=True)
    b2 = jnp.sum(gt_pos ** 2, axis=1)[None, :]
    d2 = jnp.maximum(a2 + b2 - 2.0 * pred @ gt_pos.T, 0.0)
    min_d2 = jnp.min(d2, axis=1)
    E = src.shape[0]
    eid = jnp.arange(E, dtype=jnp.int32)
    last_e = jax.ops.segment_max(eid, src, num_segments=N)
    touched = last_e >= 0
    safe_e = jnp.where(touched, last_e, 0)
    phys = jnp.where(touched, min_d2[safe_e], 0.0)
    physics_err_sq = phys[:, None]
    dist_sq = jnp.sum((pos[src_r] - lidar_pos[dst_l]) ** 2, axis=1)
    sum_d = jax.ops.segment_sum(dist_sq, src_r, num_segments=N)
    cnt_d = jax.ops.segment_sum(jnp.ones_like(dist_sq), src_r, num_segments=N)
    val = (sum_d / jnp.maximum(cnt_d, 1.0) ** 2)[:, None]
    spatial_err_sq = jnp.where(cnt_d[:, None] > 0, val, GHOST)
    denom = 2.0 * jnp.exp(lv) * node_dt ** 2 + 1e-09
    r_temp = jnp.mean(physics_err_sq / denom)
    r_spat = jnp.mean(spatial_err_sq / denom)
    r_reg = jnp.mean(0.5 * lv)
    return r_temp + r_spat + r_reg


def _finalize_kernel(parts_ref, out_ref):
    p = parts_ref[...]
    out_ref[...] = ((p[:, 0:1] + p[:, 1:2]) / N_LIDAR
                    + p[:, 2:3] / N_LIDAR + p[:, 3:4] + p[:, 4:5])


def kernel(lidar_out, lidar_pos, lidar_x, lidar_spatial_edge_index,
           radar1_out, radar1_pos, radar1_x, radar1_batch,
           radar1_temporal_edge_index, radar1_to_lidar_src,
           radar1_to_lidar_dst, radar2_out, radar2_pos, radar2_x,
           radar2_batch, radar2_temporal_edge_index, radar2_to_lidar_src,
           radar2_to_lidar_dst, dt_sec, gt_radar_pos):
    table = jnp.zeros((N_ACC, 8), jnp.float32)
    table = table.at[:N_LIDAR, 0:3].set(lidar_pos)
    table = table.at[:N_LIDAR, 3].set(lidar_x[:, 2])
    table = table.at[:N_LIDAR, 4].set(1.0)
    table = table.at[:N_LIDAR, 5].set(lidar_out[:, 0])

    src = lidar_spatial_edge_index[0].astype(jnp.int32)
    dst = lidar_spatial_edge_index[1].astype(jnp.int32)
    pad = E_PAD - E_LIDAR
    src_p = jnp.concatenate(
        [src, jnp.full((pad,), N_LIDAR, jnp.int32)]).reshape(
            2, NS, NCHUNK, NBATCH, LANE)
    dst_p = jnp.concatenate(
        [dst, jnp.full((pad,), N_ACC - 1, jnp.int32)]).reshape(
            2, NS, NCHUNK, NBATCH, LANE)
    zeros_stripe = jnp.zeros((ROWS_PER_TILE, 8), jnp.float32)

    acc = _lidar_sc(table, src_p, dst_p, zeros_stripe)

    acc0T = jnp.transpose(acc[0], (1, 0))
    acc1T = jnp.transpose(acc[1], (1, 0))
    tableT = jnp.transpose(table, (1, 0))
    lid = _lidar_tc(acc0T, acc1T, tableT)[0:1, 0:3]

    t2 = _radar_branch(radar1_out, radar1_pos, radar1_x, radar1_batch,
                       radar1_temporal_edge_index, radar1_to_lidar_src,
                       radar1_to_lidar_dst, lidar_pos, dt_sec, gt_radar_pos)
    t3 = _radar_branch(radar2_out, radar2_pos, radar2_x, radar2_batch,
                       radar2_temporal_edge_index, radar2_to_lidar_src,
                       radar2_to_lidar_dst, lidar_pos, dt_sec, gt_radar_pos)
    parts = jnp.concatenate(
        [lid, jnp.stack([t2, t3]).reshape(1, 2)], axis=1)
    out = pl.pallas_call(
        _finalize_kernel,
        out_shape=jax.ShapeDtypeStruct((1, 1), jnp.float32),
    )(parts)
    return out[0, 0]


# speedup vs baseline: 9.8352x; 2.2937x over previous
import functools
import math

import jax
import jax.numpy as jnp
from jax import lax
from jax.experimental import pallas as pl
from jax.experimental.pallas import tpu as pltpu
import jax.experimental.pallas.tpu_sc as plsc

SCALE_POSE = 10.0
SCALE_RADAR_V = 5.0
L_MIN = 2 * math.log(0.03 / SCALE_POSE + 1e-09)
L_MAX = 2 * math.log(0.5 / SCALE_POSE + 1e-09)
R_MIN = 2 * math.log(0.1 / SCALE_RADAR_V + 1e-09)
R_MAX = 2 * math.log(5.0 / SCALE_RADAR_V + 1e-09)
RADIUS_CROSS = 0.6 / SCALE_POSE
GHOST = RADIUS_CROSS ** 2

N_LIDAR = 100000
E_LIDAR = 1600000
N_ACC = 102400
NW = 32
NS = 16
LANE = 128
NBATCH = 16
NCHUNK = 25
E_TILE = NCHUNK * NBATCH * LANE
E_PAD = NW * E_TILE
ROWS_PER_TILE = N_ACC // NS


def _lidar_sc_body(table_hbm, src_hbm, dst_hbm, zeros_hbm, out_hbm,
                   idxs_v, idxd_v, rows_v, acc_sh, sem):
    c = lax.axis_index("c")
    s = lax.axis_index("s")
    pltpu.sync_copy(zeros_hbm, acc_sh.at[pl.ds(s * ROWS_PER_TILE,
                                               ROWS_PER_TILE)])
    plsc.subcore_barrier()

    def chunk(ch, _):
        pltpu.sync_copy(src_hbm.at[c, s, ch], idxs_v)
        pltpu.sync_copy(dst_hbm.at[c, s, ch], idxd_v)
        for j in range(NBATCH):
            pltpu.async_copy(table_hbm.at[idxs_v.at[j]], rows_v, sem).wait()
            pltpu.sync_copy(rows_v, acc_sh.at[idxd_v.at[j]], add=True)
        return ()

    lax.fori_loop(0, NCHUNK, chunk, ())
    plsc.subcore_barrier()
    pltpu.sync_copy(acc_sh.at[pl.ds(s * ROWS_PER_TILE, ROWS_PER_TILE)],
                    out_hbm.at[c].at[pl.ds(s * ROWS_PER_TILE, ROWS_PER_TILE)])


@functools.partial(jax.jit, static_argnums=())
def _lidar_sc(table, src_idx, dst_idx, zeros_stripe):
    mesh = plsc.VectorSubcoreMesh(core_axis_name="c", subcore_axis_name="s")
    k = pl.kernel(
        _lidar_sc_body,
        out_type=jax.ShapeDtypeStruct((2, N_ACC, 8), jnp.float32),
        mesh=mesh,
        scratch_types=[
            pltpu.VMEM((NBATCH, LANE), jnp.int32),
            pltpu.VMEM((NBATCH, LANE), jnp.int32),
            pltpu.VMEM((LANE, 8), jnp.float32),
            pltpu.VMEM_SHARED((N_ACC, 8), jnp.float32),
            pltpu.SemaphoreType.DMA,
        ],
        compiler_params=pltpu.CompilerParams(use_tc_tiling_on_sc=False, needs_layout_passes=False),
    )
    return k(table, src_idx, dst_idx, zeros_stripe)


BLK = 1024
NBLK = N_ACC // BLK


def _lidar_tc_body(acc0_ref, acc1_ref, table_ref, out_ref):
    i = pl.program_id(0)
    a = acc0_ref[...] + acc1_ref[...]
    t = table_ref[...]
    lv = jnp.clip(t[5:6, :], L_MIN, L_MAX)
    c1 = jnp.maximum(a[4:5, :], 1.0)
    mean_pos = a[0:3, :] / c1
    mean_int = a[3:4, :] / c1
    pos = t[0:3, :]
    ci = t[3:4, :]
    valid = t[4:5, :]
    res_pos = jnp.sum((pos - mean_pos) ** 2, axis=0, keepdims=True)
    res_int = (ci - mean_int) ** 2
    prec = jnp.exp(-lv)
    s1 = jnp.sum(valid * 0.5 * prec * res_pos)
    s2 = jnp.sum(valid * 0.5 * prec * res_int)
    s3 = jnp.sum(valid * 0.5 * lv)
    col = lax.broadcasted_iota(jnp.int32, (8, 128), 1)
    row = lax.broadcasted_iota(jnp.int32, (8, 128), 0)
    sel = row == 0
    part = (jnp.where(sel & (col == 0), s1, 0.0)
            + jnp.where(sel & (col == 1), s2, 0.0)
            + jnp.where(sel & (col == 2), s3, 0.0))

    @pl.when(i == 0)
    def _():
        out_ref[...] = jnp.zeros_like(out_ref)

    out_ref[...] += part


def _lidar_tc(acc0T, acc1T, tableT):
    return pl.pallas_call(
        _lidar_tc_body,
        grid=(NBLK,),
        in_specs=[
            pl.BlockSpec((8, BLK), lambda i: (0, i)),
            pl.BlockSpec((8, BLK), lambda i: (0, i)),
            pl.BlockSpec((8, BLK), lambda i: (0, i)),
        ],
        out_specs=pl.BlockSpec((8, 128), lambda i: (0, 0)),
        out_shape=jax.ShapeDtypeStruct((8, 128), jnp.float32),
    )(acc0T, acc1T, tableT)


N_RADAR = 20000
N_RACC = 20480
E_T = 100000
E_TP = 102400
T_TILE = E_TP // NS
E_C = 200000
E_CP = 204800
C_HALF = 6400
R_STRIPE = N_RACC // NS


def _radar_gather_body(rtab_hbm, ltab_hbm, tsrc_hbm, tdst_hbm, csrc_hbm,
                       cdst_hbm, s_out, d_out, cr_out, cl_out,
                       ibuf, rows_v, sem):
    c = lax.axis_index("c")
    s = lax.axis_index("s")

    def run(idx_hbm, tab_hbm, out_hbm, half):
        pltpu.sync_copy(idx_hbm, ibuf)

        def body(b, _):
            pltpu.async_copy(
                tab_hbm.at[ibuf.at[pl.ds(b * 128, 128)]], rows_v, sem).wait()
            pltpu.sync_copy(
                rows_v, out_hbm.at[pl.ds(half * C_HALF + b * 128, 128)])
            return ()
        lax.fori_loop(0, T_TILE // 128, body, ())

    run(tsrc_hbm.at[c, s], rtab_hbm,
        s_out.at[c].at[pl.ds(s * T_TILE, T_TILE)], 0)
    run(tdst_hbm.at[c, s], rtab_hbm,
        d_out.at[c].at[pl.ds(s * T_TILE, T_TILE)], 0)
    for h in range(2):
        run(csrc_hbm.at[c, s, h], rtab_hbm,
            cr_out.at[c].at[pl.ds(s * 2 * C_HALF, 2 * C_HALF)], h)
        run(cdst_hbm.at[c, s, h], ltab_hbm,
            cl_out.at[c].at[pl.ds(s * 2 * C_HALF, 2 * C_HALF)], h)


def _radar_gather(rtab, ltab, tsrc, tdst, csrc, cdst):
    mesh = plsc.VectorSubcoreMesh(core_axis_name="c", subcore_axis_name="s")
    k = pl.kernel(
        _radar_gather_body,
        out_type=(
            jax.ShapeDtypeStruct((2, E_TP, 8), jnp.float32),
            jax.ShapeDtypeStruct((2, E_TP, 8), jnp.float32),
            jax.ShapeDtypeStruct((2, E_CP, 8), jnp.float32),
            jax.ShapeDtypeStruct((2, E_CP, 8), jnp.float32),
        ),
        mesh=mesh,
        scratch_types=[
            pltpu.VMEM((T_TILE,), jnp.int32),
            pltpu.VMEM((128, 8), jnp.float32),
            pltpu.SemaphoreType.DMA,
        ],
        compiler_params=pltpu.CompilerParams(use_tc_tiling_on_sc=False, needs_layout_passes=False),
    )
    return k(rtab, ltab, tsrc, tdst, csrc, cdst)


TBLK = 2048


def _radar_temp_tc_body(st_ref, dt_ref, gt_ref, out_ref):
    sb = st_ref[...]
    db = dt_ref[...]
    g = gt_ref[...]
    move = db[0:3, :] - sb[0:3, :]
    norm = jnp.sqrt(jnp.sum(move ** 2, axis=0, keepdims=True))
    unit = move / (norm + 1e-09)
    speed = jnp.abs(sb[3:4, :])
    ndt = jnp.maximum(sb[4:5, :], 0.01)
    pred = sb[0:3, :] + speed * unit * ndt
    a2 = jnp.sum(pred ** 2, axis=0, keepdims=True)
    b2 = jnp.sum(g[0:3, :] ** 2, axis=0)
    dots = lax.dot_general(pred, g[0:3, :],
                           (((0,), (0,)), ((), ())))
    d2 = jnp.maximum(a2.reshape(TBLK, 1) + b2[None, :] - 2.0 * dots, 0.0)
    md2 = jnp.min(d2, axis=1)
    out_ref[...] = md2.reshape(16, 128)


def _radar_temp_tc(sT, dT, gt8):
    return pl.pallas_call(
        _radar_temp_tc_body,
        grid=(E_TP // TBLK,),
        in_specs=[
            pl.BlockSpec((8, TBLK), lambda i: (0, i)),
            pl.BlockSpec((8, TBLK), lambda i: (0, i)),
            pl.BlockSpec((8, 256), lambda i: (0, 0)),
        ],
        out_specs=pl.BlockSpec((16, 128), lambda i: (i, 0)),
        out_shape=jax.ShapeDtypeStruct((E_TP // 128, 128), jnp.float32),
    )(sT, dT, gt8)


def _radar_cross_tc_body(cr_ref, cl_ref, d2_ref, vl_ref):
    cr = cr_ref[...]
    cl = cl_ref[...]
    d2 = jnp.sum((cr[0:3, :] - cl[0:3, :]) ** 2, axis=0)
    vld = cr[6, :]
    d2_ref[...] = (d2 * vld).reshape(16, 128)
    vl_ref[...] = vld.reshape(16, 128)


def _radar_cross_tc(crT, clT):
    return pl.pallas_call(
        _radar_cross_tc_body,
        grid=(E_CP // TBLK,),
        in_specs=[
            pl.BlockSpec((8, TBLK), lambda i: (0, i)),
            pl.BlockSpec((8, TBLK), lambda i: (0, i)),
        ],
        out_specs=[
            pl.BlockSpec((16, 128), lambda i: (i, 0)),
            pl.BlockSpec((16, 128), lambda i: (i, 0)),
        ],
        out_shape=[
            jax.ShapeDtypeStruct((E_CP // 128, 128), jnp.float32),
            jax.ShapeDtypeStruct((E_CP // 128, 128), jnp.float32),
        ],
    )(crT, clT)


def _radar_scatter_body(tsrc_hbm, md2_hbm, cidx_hbm, d2m_hbm, vld_hbm,
                        zeros_hbm, neg_hbm, phys_out, racc_out,
                        eid_out, val_out,
                        srcbuf, md2buf, eidtab, valtab, scr16,
                        cidxbuf, d2buf, vldbuf, stage, tb, wb,
                        beste, bestv, physbuf, acc_sh):
    c = lax.axis_index("c")
    s = lax.axis_index("s")
    io = lax.iota(jnp.int32, 16)
    zeros16 = jnp.zeros((16,), jnp.int32)
    ones16 = jnp.full((16,), 1, jnp.int32)

    pltpu.sync_copy(zeros_hbm, acc_sh.at[pl.ds(s * R_STRIPE, R_STRIPE)])
    pltpu.sync_copy(neg_hbm, eidtab)
    plsc.subcore_barrier()

    pltpu.sync_copy(tsrc_hbm.at[c, s], srcbuf)
    pltpu.sync_copy(md2_hbm.at[c, s], md2buf)
    base = s * T_TILE

    def tscan(v, _):
        sv = srcbuf[pl.ds(v * 16, 16)]
        mv = md2buf[pl.ds(v * 16, 16)]
        scr16[...] = sv
        bad = sv != sv
        for k in range(1, 16):
            idxk = jnp.minimum(io + k, 15)
            sh = plsc.load_gather(scr16, [idxk])
            bad = bad | ((sh == sv) & (io < 16 - k))
        keep = jnp.logical_not(bad)
        ev = base + v * 16 + io
        plsc.store_scatter(eidtab, [sv], ev, mask=keep)
        plsc.store_scatter(valtab, [sv], mv, mask=keep)
        return ()
    lax.fori_loop(0, T_TILE // 16, tscan, ())
    pltpu.sync_copy(eidtab, eid_out.at[c, s])
    pltpu.sync_copy(valtab, val_out.at[c, s])

    def half(h):
        pltpu.sync_copy(cidx_hbm.at[c, s, h], cidxbuf)
        pltpu.sync_copy(d2m_hbm.at[c, s, h], d2buf)
        pltpu.sync_copy(vld_hbm.at[c, s, h], vldbuf)

        def batch(g, _):
            for j in range(8):
                d2v = d2buf[pl.ds(g * 128 + j * 16, 16)]
                vlv = vldbuf[pl.ds(g * 128 + j * 16, 16)]
                rows16 = j * 16 + io
                plsc.store_scatter(stage, [rows16, zeros16], d2v)
                plsc.store_scatter(stage, [rows16, ones16], vlv)
            pltpu.sync_copy(stage, acc_sh.at[cidxbuf.at[g]], add=True)
            return ()
        lax.fori_loop(0, 50, batch, ())
    half(0)
    half(1)
    plsc.subcore_barrier()

    for t in range(NS):
        pltpu.sync_copy(eid_out.at[c, t].at[pl.ds(s * R_STRIPE, R_STRIPE)],
                        tb)
        pltpu.sync_copy(val_out.at[c, t].at[pl.ds(s * R_STRIPE, R_STRIPE)],
                        wb)
        if t == 0:
            def init(v, _):
                beste[pl.ds(v * 16, 16)] = tb[pl.ds(v * 16, 16)]
                bestv[pl.ds(v * 16, 16)] = wb[pl.ds(v * 16, 16)]
                return ()
            lax.fori_loop(0, R_STRIPE // 16, init, ())
        else:
            def upd(v, _):
                et = tb[pl.ds(v * 16, 16)]
                vt = wb[pl.ds(v * 16, 16)]
                be = beste[pl.ds(v * 16, 16)]
                m = et > be
                beste[pl.ds(v * 16, 16)] = jnp.where(m, et, be)
                bestv[pl.ds(v * 16, 16)] = jnp.where(
                    m, vt, bestv[pl.ds(v * 16, 16)])
                return ()
            lax.fori_loop(0, R_STRIPE // 16, upd, ())

    def finph(v, _):
        physbuf[pl.ds(v * 16, 16)] = jnp.where(
            beste[pl.ds(v * 16, 16)] >= 0, bestv[pl.ds(v * 16, 16)],
            jnp.zeros((16,), jnp.float32))
        return ()
    lax.fori_loop(0, R_STRIPE // 16, finph, ())
    pltpu.sync_copy(physbuf, phys_out.at[c].at[pl.ds(s * R_STRIPE, R_STRIPE)])
    pltpu.sync_copy(acc_sh.at[pl.ds(s * R_STRIPE, R_STRIPE)],
                    racc_out.at[c].at[pl.ds(s * R_STRIPE, R_STRIPE)])


def _radar_scatter(tsrc, md2, cidx, d2m, vld, zeros_stripe, neg1):
    mesh = plsc.VectorSubcoreMesh(core_axis_name="c", subcore_axis_name="s")
    k = pl.kernel(
        _radar_scatter_body,
        out_type=(
            jax.ShapeDtypeStruct((2, N_RACC), jnp.float32),
            jax.ShapeDtypeStruct((2, N_RACC, 8), jnp.float32),
            jax.ShapeDtypeStruct((2, NS, N_RACC), jnp.int32),
            jax.ShapeDtypeStruct((2, NS, N_RACC), jnp.float32),
        ),
        mesh=mesh,
        scratch_types=[
            pltpu.VMEM((T_TILE,), jnp.int32),
            pltpu.VMEM((T_TILE,), jnp.float32),
            pltpu.VMEM((N_RACC,), jnp.int32),
            pltpu.VMEM((N_RACC,), jnp.float32),
            pltpu.VMEM((16,), jnp.int32),
            pltpu.VMEM((50, 128), jnp.int32),
            pltpu.VMEM((C_HALF,), jnp.float32),
            pltpu.VMEM((C_HALF,), jnp.float32),
            pltpu.VMEM((128, 8), jnp.float32),
            pltpu.VMEM((R_STRIPE,), jnp.int32),
            pltpu.VMEM((R_STRIPE,), jnp.float32),
            pltpu.VMEM((R_STRIPE,), jnp.int32),
            pltpu.VMEM((R_STRIPE,), jnp.float32),
            pltpu.VMEM((R_STRIPE,), jnp.float32),
            pltpu.VMEM_SHARED((N_RACC, 8), jnp.float32),
        ],
        compiler_params=pltpu.CompilerParams(use_tc_tiling_on_sc=False, needs_layout_passes=False),
    )
    return k(tsrc, md2, cidx, d2m, vld, zeros_stripe, neg1)


def _final_tc_body(lid_ref, lv1, dt1, vl1, sd1, ct1, ph1,
                   lv2, dt2, vl2, sd2, ct2, ph2, out_ref):
    lid = lid_ref[...]
    total = (lid[0, 0] + lid[0, 1] + lid[0, 2]) / N_LIDAR

    def branch(lv_r, dt_r, vl_r, sd_r, ct_r, ph_r):
        lv = jnp.clip(lv_r[...], R_MIN, R_MAX)
        ndt = jnp.maximum(dt_r[...], 0.01)
        vld = vl_r[...]
        sum_d = sd_r[...]
        cnt = ct_r[...]
        phys = ph_r[...]
        spat = jnp.where(cnt > 0,
                         sum_d / jnp.maximum(cnt, 1.0) ** 2, GHOST)
        denom = 2.0 * jnp.exp(lv) * ndt ** 2 + 1e-09
        return jnp.sum(vld * (phys / denom + spat / denom + 0.5 * lv))

    total = total + branch(lv1, dt1, vl1, sd1, ct1, ph1) / N_RADAR
    total = total + branch(lv2, dt2, vl2, sd2, ct2, ph2) / N_RADAR
    out_ref[...] = total.reshape(1, 1)


def _final_tc(lid, args1, args2):
    return pl.pallas_call(
        _final_tc_body,
        out_shape=jax.ShapeDtypeStruct((1, 1), jnp.float32),
    )(lid, *args1, *args2)


def kernel(lidar_out, lidar_pos, lidar_x, lidar_spatial_edge_index,
           radar1_out, radar1_pos, radar1_x, radar1_batch,
           radar1_temporal_edge_index, radar1_to_lidar_src,
           radar1_to_lidar_dst, radar2_out, radar2_pos, radar2_x,
           radar2_batch, radar2_temporal_edge_index, radar2_to_lidar_src,
           radar2_to_lidar_dst, dt_sec, gt_radar_pos):
    table = jnp.zeros((N_ACC, 8), jnp.float32)
    table = table.at[:N_LIDAR, 0:3].set(lidar_pos)
    table = table.at[:N_LIDAR, 3].set(lidar_x[:, 2])
    table = table.at[:N_LIDAR, 4].set(1.0)
    table = table.at[:N_LIDAR, 5].set(lidar_out[:, 0])

    src = lidar_spatial_edge_index[0].astype(jnp.int32)
    dst = lidar_spatial_edge_index[1].astype(jnp.int32)
    pad = E_PAD - E_LIDAR
    src_p = jnp.concatenate(
        [src, jnp.full((pad,), N_LIDAR, jnp.int32)]).reshape(
            2, NS, NCHUNK, NBATCH, LANE)
    dst_p = jnp.concatenate(
        [dst, jnp.full((pad,), N_ACC - 1, jnp.int32)]).reshape(
            2, NS, NCHUNK, NBATCH, LANE)
    zeros_stripe = jnp.zeros((ROWS_PER_TILE, 8), jnp.float32)

    acc = _lidar_sc(table, src_p, dst_p, zeros_stripe)

    acc0T = jnp.transpose(acc[0], (1, 0))
    acc1T = jnp.transpose(acc[1], (1, 0))
    tableT = jnp.transpose(table, (1, 0))
    lid = _lidar_tc(acc0T, acc1T, tableT)[0:1, :]

    def prep_branch(out_r, pos_r, x_r, batch_r, tei, srcr, dstl):
        rows = jnp.zeros((N_RACC, 8), jnp.float32)
        rows = rows.at[:N_RADAR, 0:3].set(pos_r)
        rows = rows.at[:N_RADAR, 3].set(x_r[:, 2])
        rows = rows.at[:N_RADAR, 4].set(jnp.take(dt_sec, batch_r))
        rows = rows.at[:N_RADAR, 5].set(out_r[:, 0])
        rows = rows.at[:N_RADAR, 6].set(1.0)
        tpad = jnp.full((E_TP - E_T,), N_RADAR, jnp.int32)
        cpad = jnp.full((E_CP - E_C,), N_RADAR, jnp.int32)
        ts = jnp.concatenate([tei[0].astype(jnp.int32), tpad])
        td = jnp.concatenate([tei[1].astype(jnp.int32), tpad])
        cs = jnp.concatenate([srcr.astype(jnp.int32), cpad])
        cd = jnp.concatenate([dstl.astype(jnp.int32),
                              jnp.full((E_CP - E_C,), N_LIDAR, jnp.int32)])
        return rows, ts, td, cs, cd

    b1 = prep_branch(radar1_out, radar1_pos, radar1_x, radar1_batch,
                     radar1_temporal_edge_index, radar1_to_lidar_src,
                     radar1_to_lidar_dst)
    b2 = prep_branch(radar2_out, radar2_pos, radar2_x, radar2_batch,
                     radar2_temporal_edge_index, radar2_to_lidar_src,
                     radar2_to_lidar_dst)
    rtab = jnp.concatenate([b1[0], b2[0]], axis=0)
    tsrc_l = jnp.stack([b1[1], b2[1]]).reshape(2, NS, T_TILE)
    tsrc_g = jnp.stack([b1[1], b2[1] + N_RACC]).reshape(2, NS, T_TILE)
    tdst_g = jnp.stack([b1[2], b2[2] + N_RACC]).reshape(2, NS, T_TILE)
    csrc_l = jnp.stack([b1[3], b2[3]]).reshape(2, NS, 2, 50, 128)
    csrc_g = jnp.stack([b1[3], b2[3] + N_RACC]).reshape(2, NS, 2, C_HALF)
    cdst_g = jnp.stack([b1[4], b2[4]]).reshape(2, NS, 2, C_HALF)

    S, D, Cr, Cl = _radar_gather(rtab, table, tsrc_g, tdst_g, csrc_g, cdst_g)

    gt8 = jnp.zeros((8, 256), jnp.float32).at[0:3, :].set(gt_radar_pos.T)
    md2_1 = _radar_temp_tc(S[0].T, D[0].T, gt8)
    md2_2 = _radar_temp_tc(S[1].T, D[1].T, gt8)
    d2m_1, vld_1 = _radar_cross_tc(Cr[0].T, Cl[0].T)
    d2m_2, vld_2 = _radar_cross_tc(Cr[1].T, Cl[1].T)

    md2 = jnp.stack([md2_1.reshape(NS, T_TILE), md2_2.reshape(NS, T_TILE)])
    d2m = jnp.stack([d2m_1.reshape(NS, 2, C_HALF),
                     d2m_2.reshape(NS, 2, C_HALF)])
    vld = jnp.stack([vld_1.reshape(NS, 2, C_HALF),
                     vld_2.reshape(NS, 2, C_HALF)])
    zeros_r = jnp.zeros((R_STRIPE, 8), jnp.float32)
    neg1 = jnp.full((N_RACC,), -1, jnp.int32)
    phys, racc, _eid, _val = _radar_scatter(tsrc_l, md2, csrc_l, d2m, vld,
                                            zeros_r, neg1)

    def final_args(b, rows):
        return (rows[:, 5].reshape(1, N_RACC), rows[:, 4].reshape(1, N_RACC),
                rows[:, 6].reshape(1, N_RACC),
                racc[b, :, 0].reshape(1, N_RACC),
                racc[b, :, 1].reshape(1, N_RACC),
                phys[b].reshape(1, N_RACC))

    out = _final_tc(lid, final_args(0, b1[0]), final_args(1, b2[0]))
    return out[0, 0]


# speedup vs baseline: 9.8416x; 1.0006x over previous
import functools
import math

import jax
import jax.numpy as jnp
from jax import lax
from jax.experimental import pallas as pl
from jax.experimental.pallas import tpu as pltpu
import jax.experimental.pallas.tpu_sc as plsc

SCALE_POSE = 10.0
SCALE_RADAR_V = 5.0
L_MIN = 2 * math.log(0.03 / SCALE_POSE + 1e-09)
L_MAX = 2 * math.log(0.5 / SCALE_POSE + 1e-09)
R_MIN = 2 * math.log(0.1 / SCALE_RADAR_V + 1e-09)
R_MAX = 2 * math.log(5.0 / SCALE_RADAR_V + 1e-09)
RADIUS_CROSS = 0.6 / SCALE_POSE
GHOST = RADIUS_CROSS ** 2

N_LIDAR = 100000
E_LIDAR = 1600000
N_ACC = 102400
NW = 32
NS = 16
LANE = 128
NBATCH = 16
NCHUNK = 25
E_TILE = NCHUNK * NBATCH * LANE
E_PAD = NW * E_TILE
ROWS_PER_TILE = N_ACC // NS


def _lidar_sc_body(table_hbm, src_hbm, dst_hbm, zeros_hbm, out_hbm,
                   idxs_v, idxd_v, rows_v, acc_sh, sems):
    c = lax.axis_index("c")
    s = lax.axis_index("s")
    pltpu.sync_copy(zeros_hbm, acc_sh.at[pl.ds(s * ROWS_PER_TILE,
                                               ROWS_PER_TILE)])
    plsc.subcore_barrier()

    def chunk(ch, _):
        pltpu.sync_copy(src_hbm.at[c, s, ch], idxs_v)
        pltpu.sync_copy(dst_hbm.at[c, s, ch], idxd_v)
        descs = [None, None]
        descs[0] = pltpu.async_copy(table_hbm.at[idxs_v.at[0]],
                                    rows_v.at[0], sems.at[0])
        for j in range(NBATCH):
            descs[j % 2].wait()
            if j + 1 < NBATCH:
                descs[(j + 1) % 2] = pltpu.async_copy(
                    table_hbm.at[idxs_v.at[j + 1]],
                    rows_v.at[(j + 1) % 2], sems.at[(j + 1) % 2])
            pltpu.sync_copy(rows_v.at[j % 2], acc_sh.at[idxd_v.at[j]],
                            add=True)
        return ()

    lax.fori_loop(0, NCHUNK, chunk, ())
    plsc.subcore_barrier()
    pltpu.sync_copy(acc_sh.at[pl.ds(s * ROWS_PER_TILE, ROWS_PER_TILE)],
                    out_hbm.at[c].at[pl.ds(s * ROWS_PER_TILE, ROWS_PER_TILE)])


@functools.partial(jax.jit, static_argnums=())
def _lidar_sc(table, src_idx, dst_idx, zeros_stripe):
    mesh = plsc.VectorSubcoreMesh(core_axis_name="c", subcore_axis_name="s")
    k = pl.kernel(
        _lidar_sc_body,
        out_type=jax.ShapeDtypeStruct((2, N_ACC, 8), jnp.float32),
        mesh=mesh,
        scratch_types=[
            pltpu.VMEM((NBATCH, LANE), jnp.int32),
            pltpu.VMEM((NBATCH, LANE), jnp.int32),
            pltpu.VMEM((2, LANE, 8), jnp.float32),
            pltpu.VMEM_SHARED((N_ACC, 8), jnp.float32),
            pltpu.SemaphoreType.DMA((2,)),
        ],
        compiler_params=pltpu.CompilerParams(use_tc_tiling_on_sc=False, needs_layout_passes=False),
    )
    return k(table, src_idx, dst_idx, zeros_stripe)


BLK = 1024
NBLK = N_ACC // BLK


def _lidar_tc_body(acc0_ref, acc1_ref, table_ref, out_ref):
    i = pl.program_id(0)
    a = acc0_ref[...] + acc1_ref[...]
    t = table_ref[...]
    lv = jnp.clip(t[5:6, :], L_MIN, L_MAX)
    c1 = jnp.maximum(a[4:5, :], 1.0)
    mean_pos = a[0:3, :] / c1
    mean_int = a[3:4, :] / c1
    pos = t[0:3, :]
    ci = t[3:4, :]
    valid = t[4:5, :]
    res_pos = jnp.sum((pos - mean_pos) ** 2, axis=0, keepdims=True)
    res_int = (ci - mean_int) ** 2
    prec = jnp.exp(-lv)
    s1 = jnp.sum(valid * 0.5 * prec * res_pos)
    s2 = jnp.sum(valid * 0.5 * prec * res_int)
    s3 = jnp.sum(valid * 0.5 * lv)
    col = lax.broadcasted_iota(jnp.int32, (8, 128), 1)
    row = lax.broadcasted_iota(jnp.int32, (8, 128), 0)
    sel = row == 0
    part = (jnp.where(sel & (col == 0), s1, 0.0)
            + jnp.where(sel & (col == 1), s2, 0.0)
            + jnp.where(sel & (col == 2), s3, 0.0))

    @pl.when(i == 0)
    def _():
        out_ref[...] = jnp.zeros_like(out_ref)

    out_ref[...] += part


def _lidar_tc(acc0T, acc1T, tableT):
    return pl.pallas_call(
        _lidar_tc_body,
        grid=(NBLK,),
        in_specs=[
            pl.BlockSpec((8, BLK), lambda i: (0, i)),
            pl.BlockSpec((8, BLK), lambda i: (0, i)),
            pl.BlockSpec((8, BLK), lambda i: (0, i)),
        ],
        out_specs=pl.BlockSpec((8, 128), lambda i: (0, 0)),
        out_shape=jax.ShapeDtypeStruct((8, 128), jnp.float32),
    )(acc0T, acc1T, tableT)


N_RADAR = 20000
N_RACC = 20480
E_T = 100000
E_TP = 102400
T_TILE = E_TP // NS
E_C = 200000
E_CP = 204800
C_HALF = 6400
R_STRIPE = N_RACC // NS


def _radar_gather_body(rtab_hbm, ltab_hbm, tsrc_hbm, tdst_hbm, csrc_hbm,
                       cdst_hbm, s_out, d_out, cr_out, cl_out,
                       ibuf, rows_v, sem):
    c = lax.axis_index("c")
    s = lax.axis_index("s")

    def run(idx_hbm, tab_hbm, out_hbm, half):
        pltpu.sync_copy(idx_hbm, ibuf)

        def body(b, _):
            pltpu.async_copy(
                tab_hbm.at[ibuf.at[pl.ds(b * 128, 128)]], rows_v, sem).wait()
            pltpu.sync_copy(
                rows_v, out_hbm.at[pl.ds(half * C_HALF + b * 128, 128)])
            return ()
        lax.fori_loop(0, T_TILE // 128, body, ())

    run(tsrc_hbm.at[c, s], rtab_hbm,
        s_out.at[c].at[pl.ds(s * T_TILE, T_TILE)], 0)
    run(tdst_hbm.at[c, s], rtab_hbm,
        d_out.at[c].at[pl.ds(s * T_TILE, T_TILE)], 0)
    for h in range(2):
        run(csrc_hbm.at[c, s, h], rtab_hbm,
            cr_out.at[c].at[pl.ds(s * 2 * C_HALF, 2 * C_HALF)], h)
        run(cdst_hbm.at[c, s, h], ltab_hbm,
            cl_out.at[c].at[pl.ds(s * 2 * C_HALF, 2 * C_HALF)], h)


def _radar_gather(rtab, ltab, tsrc, tdst, csrc, cdst):
    mesh = plsc.VectorSubcoreMesh(core_axis_name="c", subcore_axis_name="s")
    k = pl.kernel(
        _radar_gather_body,
        out_type=(
            jax.ShapeDtypeStruct((2, E_TP, 8), jnp.float32),
            jax.ShapeDtypeStruct((2, E_TP, 8), jnp.float32),
            jax.ShapeDtypeStruct((2, E_CP, 8), jnp.float32),
            jax.ShapeDtypeStruct((2, E_CP, 8), jnp.float32),
        ),
        mesh=mesh,
        scratch_types=[
            pltpu.VMEM((T_TILE,), jnp.int32),
            pltpu.VMEM((128, 8), jnp.float32),
            pltpu.SemaphoreType.DMA,
        ],
        compiler_params=pltpu.CompilerParams(use_tc_tiling_on_sc=False, needs_layout_passes=False),
    )
    return k(rtab, ltab, tsrc, tdst, csrc, cdst)


TBLK = 2048


def _radar_temp_tc_body(st_ref, dt_ref, gt_ref, out_ref):
    sb = st_ref[...]
    db = dt_ref[...]
    g = gt_ref[...]
    move = db[0:3, :] - sb[0:3, :]
    norm = jnp.sqrt(jnp.sum(move ** 2, axis=0, keepdims=True))
    unit = move / (norm + 1e-09)
    speed = jnp.abs(sb[3:4, :])
    ndt = jnp.maximum(sb[4:5, :], 0.01)
    pred = sb[0:3, :] + speed * unit * ndt
    a2 = jnp.sum(pred ** 2, axis=0, keepdims=True)
    b2 = jnp.sum(g[0:3, :] ** 2, axis=0)
    dots = lax.dot_general(pred, g[0:3, :],
                           (((0,), (0,)), ((), ())))
    d2 = jnp.maximum(a2.reshape(TBLK, 1) + b2[None, :] - 2.0 * dots, 0.0)
    md2 = jnp.min(d2, axis=1)
    out_ref[...] = md2.reshape(16, 128)


def _radar_temp_tc(sT, dT, gt8):
    return pl.pallas_call(
        _radar_temp_tc_body,
        grid=(E_TP // TBLK,),
        in_specs=[
            pl.BlockSpec((8, TBLK), lambda i: (0, i)),
            pl.BlockSpec((8, TBLK), lambda i: (0, i)),
            pl.BlockSpec((8, 256), lambda i: (0, 0)),
        ],
        out_specs=pl.BlockSpec((16, 128), lambda i: (i, 0)),
        out_shape=jax.ShapeDtypeStruct((E_TP // 128, 128), jnp.float32),
    )(sT, dT, gt8)


def _radar_cross_tc_body(cr_ref, cl_ref, d2_ref, vl_ref):
    cr = cr_ref[...]
    cl = cl_ref[...]
    d2 = jnp.sum((cr[0:3, :] - cl[0:3, :]) ** 2, axis=0)
    vld = cr[6, :]
    d2_ref[...] = (d2 * vld).reshape(16, 128)
    vl_ref[...] = vld.reshape(16, 128)


def _radar_cross_tc(crT, clT):
    return pl.pallas_call(
        _radar_cross_tc_body,
        grid=(E_CP // TBLK,),
        in_specs=[
            pl.BlockSpec((8, TBLK), lambda i: (0, i)),
            pl.BlockSpec((8, TBLK), lambda i: (0, i)),
        ],
        out_specs=[
            pl.BlockSpec((16, 128), lambda i: (i, 0)),
            pl.BlockSpec((16, 128), lambda i: (i, 0)),
        ],
        out_shape=[
            jax.ShapeDtypeStruct((E_CP // 128, 128), jnp.float32),
            jax.ShapeDtypeStruct((E_CP // 128, 128), jnp.float32),
        ],
    )(crT, clT)


def _radar_scatter_body(tsrc_hbm, md2_hbm, cidx_hbm, d2m_hbm, vld_hbm,
                        zeros_hbm, neg_hbm, phys_out, racc_out,
                        eid_out, val_out,
                        srcbuf, md2buf, eidtab, valtab, scr16,
                        cidxbuf, d2buf, vldbuf, stage, tb, wb,
                        beste, bestv, physbuf, acc_sh):
    c = lax.axis_index("c")
    s = lax.axis_index("s")
    io = lax.iota(jnp.int32, 16)
    zeros16 = jnp.zeros((16,), jnp.int32)
    ones16 = jnp.full((16,), 1, jnp.int32)

    pltpu.sync_copy(zeros_hbm, acc_sh.at[pl.ds(s * R_STRIPE, R_STRIPE)])
    pltpu.sync_copy(neg_hbm, eidtab)
    plsc.subcore_barrier()

    pltpu.sync_copy(tsrc_hbm.at[c, s], srcbuf)
    pltpu.sync_copy(md2_hbm.at[c, s], md2buf)
    base = s * T_TILE

    def tscan(v, _):
        sv = srcbuf[pl.ds(v * 16, 16)]
        mv = md2buf[pl.ds(v * 16, 16)]
        scr16[...] = sv
        bad = sv != sv
        for k in range(1, 16):
            idxk = jnp.minimum(io + k, 15)
            sh = plsc.load_gather(scr16, [idxk])
            bad = bad | ((sh == sv) & (io < 16 - k))
        keep = jnp.logical_not(bad)
        ev = base + v * 16 + io
        plsc.store_scatter(eidtab, [sv], ev, mask=keep)
        plsc.store_scatter(valtab, [sv], mv, mask=keep)
        return ()
    lax.fori_loop(0, T_TILE // 16, tscan, ())
    pltpu.sync_copy(eidtab, eid_out.at[c, s])
    pltpu.sync_copy(valtab, val_out.at[c, s])

    def half(h):
        pltpu.sync_copy(cidx_hbm.at[c, s, h], cidxbuf)
        pltpu.sync_copy(d2m_hbm.at[c, s, h], d2buf)
        pltpu.sync_copy(vld_hbm.at[c, s, h], vldbuf)

        def batch(g, _):
            for j in range(8):
                d2v = d2buf[pl.ds(g * 128 + j * 16, 16)]
                vlv = vldbuf[pl.ds(g * 128 + j * 16, 16)]
                rows16 = j * 16 + io
                plsc.store_scatter(stage, [rows16, zeros16], d2v)
                plsc.store_scatter(stage, [rows16, ones16], vlv)
            pltpu.sync_copy(stage, acc_sh.at[cidxbuf.at[g]], add=True)
            return ()
        lax.fori_loop(0, 50, batch, ())
    half(0)
    half(1)
    plsc.subcore_barrier()

    for t in range(NS):
        pltpu.sync_copy(eid_out.at[c, t].at[pl.ds(s * R_STRIPE, R_STRIPE)],
                        tb)
        pltpu.sync_copy(val_out.at[c, t].at[pl.ds(s * R_STRIPE, R_STRIPE)],
                        wb)
        if t == 0:
            def init(v, _):
                beste[pl.ds(v * 16, 16)] = tb[pl.ds(v * 16, 16)]
                bestv[pl.ds(v * 16, 16)] = wb[pl.ds(v * 16, 16)]
                return ()
            lax.fori_loop(0, R_STRIPE // 16, init, ())
        else:
            def upd(v, _):
                et = tb[pl.ds(v * 16, 16)]
                vt = wb[pl.ds(v * 16, 16)]
                be = beste[pl.ds(v * 16, 16)]
                m = et > be
                beste[pl.ds(v * 16, 16)] = jnp.where(m, et, be)
                bestv[pl.ds(v * 16, 16)] = jnp.where(
                    m, vt, bestv[pl.ds(v * 16, 16)])
                return ()
            lax.fori_loop(0, R_STRIPE // 16, upd, ())

    def finph(v, _):
        physbuf[pl.ds(v * 16, 16)] = jnp.where(
            beste[pl.ds(v * 16, 16)] >= 0, bestv[pl.ds(v * 16, 16)],
            jnp.zeros((16,), jnp.float32))
        return ()
    lax.fori_loop(0, R_STRIPE // 16, finph, ())
    pltpu.sync_copy(physbuf, phys_out.at[c].at[pl.ds(s * R_STRIPE, R_STRIPE)])
    pltpu.sync_copy(acc_sh.at[pl.ds(s * R_STRIPE, R_STRIPE)],
                    racc_out.at[c].at[pl.ds(s * R_STRIPE, R_STRIPE)])


def _radar_scatter(tsrc, md2, cidx, d2m, vld, zeros_stripe, neg1):
    mesh = plsc.VectorSubcoreMesh(core_axis_name="c", subcore_axis_name="s")
    k = pl.kernel(
        _radar_scatter_body,
        out_type=(
            jax.ShapeDtypeStruct((2, N_RACC), jnp.float32),
            jax.ShapeDtypeStruct((2, N_RACC, 8), jnp.float32),
            jax.ShapeDtypeStruct((2, NS, N_RACC), jnp.int32),
            jax.ShapeDtypeStruct((2, NS, N_RACC), jnp.float32),
        ),
        mesh=mesh,
        scratch_types=[
            pltpu.VMEM((T_TILE,), jnp.int32),
            pltpu.VMEM((T_TILE,), jnp.float32),
            pltpu.VMEM((N_RACC,), jnp.int32),
            pltpu.VMEM((N_RACC,), jnp.float32),
            pltpu.VMEM((16,), jnp.int32),
            pltpu.VMEM((50, 128), jnp.int32),
            pltpu.VMEM((C_HALF,), jnp.float32),
            pltpu.VMEM((C_HALF,), jnp.float32),
            pltpu.VMEM((128, 8), jnp.float32),
            pltpu.VMEM((R_STRIPE,), jnp.int32),
            pltpu.VMEM((R_STRIPE,), jnp.float32),
            pltpu.VMEM((R_STRIPE,), jnp.int32),
            pltpu.VMEM((R_STRIPE,), jnp.float32),
            pltpu.VMEM((R_STRIPE,), jnp.float32),
            pltpu.VMEM_SHARED((N_RACC, 8), jnp.float32),
        ],
        compiler_params=pltpu.CompilerParams(use_tc_tiling_on_sc=False, needs_layout_passes=False),
    )
    return k(tsrc, md2, cidx, d2m, vld, zeros_stripe, neg1)


def _final_tc_body(lid_ref, lv1, dt1, vl1, sd1, ct1, ph1,
                   lv2, dt2, vl2, sd2, ct2, ph2, out_ref):
    lid = lid_ref[...]
    total = (lid[0, 0] + lid[0, 1] + lid[0, 2]) / N_LIDAR

    def branch(lv_r, dt_r, vl_r, sd_r, ct_r, ph_r):
        lv = jnp.clip(lv_r[...], R_MIN, R_MAX)
        ndt = jnp.maximum(dt_r[...], 0.01)
        vld = vl_r[...]
        sum_d = sd_r[...]
        cnt = ct_r[...]
        phys = ph_r[...]
        spat = jnp.where(cnt > 0,
                         sum_d / jnp.maximum(cnt, 1.0) ** 2, GHOST)
        denom = 2.0 * jnp.exp(lv) * ndt ** 2 + 1e-09
        return jnp.sum(vld * (phys / denom + spat / denom + 0.5 * lv))

    total = total + branch(lv1, dt1, vl1, sd1, ct1, ph1) / N_RADAR
    total = total + branch(lv2, dt2, vl2, sd2, ct2, ph2) / N_RADAR
    out_ref[...] = total.reshape(1, 1)


def _final_tc(lid, args1, args2):
    return pl.pallas_call(
        _final_tc_body,
        out_shape=jax.ShapeDtypeStruct((1, 1), jnp.float32),
    )(lid, *args1, *args2)


def kernel(lidar_out, lidar_pos, lidar_x, lidar_spatial_edge_index,
           radar1_out, radar1_pos, radar1_x, radar1_batch,
           radar1_temporal_edge_index, radar1_to_lidar_src,
           radar1_to_lidar_dst, radar2_out, radar2_pos, radar2_x,
           radar2_batch, radar2_temporal_edge_index, radar2_to_lidar_src,
           radar2_to_lidar_dst, dt_sec, gt_radar_pos):
    table = jnp.zeros((N_ACC, 8), jnp.float32)
    table = table.at[:N_LIDAR, 0:3].set(lidar_pos)
    table = table.at[:N_LIDAR, 3].set(lidar_x[:, 2])
    table = table.at[:N_LIDAR, 4].set(1.0)
    table = table.at[:N_LIDAR, 5].set(lidar_out[:, 0])

    src = lidar_spatial_edge_index[0].astype(jnp.int32)
    dst = lidar_spatial_edge_index[1].astype(jnp.int32)
    pad = E_PAD - E_LIDAR
    src_p = jnp.concatenate(
        [src, jnp.full((pad,), N_LIDAR, jnp.int32)]).reshape(
            2, NS, NCHUNK, NBATCH, LANE)
    dst_p = jnp.concatenate(
        [dst, jnp.full((pad,), N_ACC - 1, jnp.int32)]).reshape(
            2, NS, NCHUNK, NBATCH, LANE)
    zeros_stripe = jnp.zeros((ROWS_PER_TILE, 8), jnp.float32)

    acc = _lidar_sc(table, src_p, dst_p, zeros_stripe)

    acc0T = jnp.transpose(acc[0], (1, 0))
    acc1T = jnp.transpose(acc[1], (1, 0))
    tableT = jnp.transpose(table, (1, 0))
    lid = _lidar_tc(acc0T, acc1T, tableT)[0:1, :]

    def prep_branch(out_r, pos_r, x_r, batch_r, tei, srcr, dstl):
        rows = jnp.zeros((N_RACC, 8), jnp.float32)
        rows = rows.at[:N_RADAR, 0:3].set(pos_r)
        rows = rows.at[:N_RADAR, 3].set(x_r[:, 2])
        rows = rows.at[:N_RADAR, 4].set(jnp.take(dt_sec, batch_r))
        rows = rows.at[:N_RADAR, 5].set(out_r[:, 0])
        rows = rows.at[:N_RADAR, 6].set(1.0)
        tpad = jnp.full((E_TP - E_T,), N_RADAR, jnp.int32)
        cpad = jnp.full((E_CP - E_C,), N_RADAR, jnp.int32)
        ts = jnp.concatenate([tei[0].astype(jnp.int32), tpad])
        td = jnp.concatenate([tei[1].astype(jnp.int32), tpad])
        cs = jnp.concatenate([srcr.astype(jnp.int32), cpad])
        cd = jnp.concatenate([dstl.astype(jnp.int32),
                              jnp.full((E_CP - E_C,), N_LIDAR, jnp.int32)])
        return rows, ts, td, cs, cd

    b1 = prep_branch(radar1_out, radar1_pos, radar1_x, radar1_batch,
                     radar1_temporal_edge_index, radar1_to_lidar_src,
                     radar1_to_lidar_dst)
    b2 = prep_branch(radar2_out, radar2_pos, radar2_x, radar2_batch,
                     radar2_temporal_edge_index, radar2_to_lidar_src,
                     radar2_to_lidar_dst)
    rtab = jnp.concatenate([b1[0], b2[0]], axis=0)
    tsrc_l = jnp.stack([b1[1], b2[1]]).reshape(2, NS, T_TILE)
    tsrc_g = jnp.stack([b1[1], b2[1] + N_RACC]).reshape(2, NS, T_TILE)
    tdst_g = jnp.stack([b1[2], b2[2] + N_RACC]).reshape(2, NS, T_TILE)
    csrc_l = jnp.stack([b1[3], b2[3]]).reshape(2, NS, 2, 50, 128)
    csrc_g = jnp.stack([b1[3], b2[3] + N_RACC]).reshape(2, NS, 2, C_HALF)
    cdst_g = jnp.stack([b1[4], b2[4]]).reshape(2, NS, 2, C_HALF)

    S, D, Cr, Cl = _radar_gather(rtab, table, tsrc_g, tdst_g, csrc_g, cdst_g)

    gt8 = jnp.zeros((8, 256), jnp.float32).at[0:3, :].set(gt_radar_pos.T)
    md2_1 = _radar_temp_tc(S[0].T, D[0].T, gt8)
    md2_2 = _radar_temp_tc(S[1].T, D[1].T, gt8)
    d2m_1, vld_1 = _radar_cross_tc(Cr[0].T, Cl[0].T)
    d2m_2, vld_2 = _radar_cross_tc(Cr[1].T, Cl[1].T)

    md2 = jnp.stack([md2_1.reshape(NS, T_TILE), md2_2.reshape(NS, T_TILE)])
    d2m = jnp.stack([d2m_1.reshape(NS, 2, C_HALF),
                     d2m_2.reshape(NS, 2, C_HALF)])
    vld = jnp.stack([vld_1.reshape(NS, 2, C_HALF),
                     vld_2.reshape(NS, 2, C_HALF)])
    zeros_r = jnp.zeros((R_STRIPE, 8), jnp.float32)
    neg1 = jnp.full((N_RACC,), -1, jnp.int32)
    phys, racc, _eid, _val = _radar_scatter(tsrc_l, md2, csrc_l, d2m, vld,
                                            zeros_r, neg1)

    def final_args(b, rows):
        return (rows[:, 5].reshape(1, N_RACC), rows[:, 4].reshape(1, N_RACC),
                rows[:, 6].reshape(1, N_RACC),
                racc[b, :, 0].reshape(1, N_RACC),
                racc[b, :, 1].reshape(1, N_RACC),
                phys[b].reshape(1, N_RACC))

    out = _final_tc(lid, final_args(0, b1[0]), final_args(1, b2[0]))
    return out[0, 0]
